# Initial kernel scaffold; baseline (speedup 1.0000x reference)
import jax
import jax.numpy as jnp
from jax.experimental import pallas as pl


def kernel(x, edge_index, adj_vals, W):
    raise NotImplementedError("write your pallas kernel here")


# speedup vs baseline: 3.8937x; 3.8937x over previous
import functools

import jax
import jax.numpy as jnp
from jax import lax
from jax.experimental import pallas as pl
from jax.experimental.pallas import tpu as pltpu
from jax.experimental.pallas import tpu_sc as plsc

N = 10000
D = 128
NC = 2
NS = 16
NW = NC * NS
L = 16
CHUNK = 128
ROWS_PER_TILE = 640
N_PAD = NS * ROWS_PER_TILE


def _matmul(x, W):
    def body(x_ref, w_ref, o_ref):
        o_ref[...] = jnp.dot(x_ref[...], w_ref[...],
                             preferred_element_type=jnp.float32)

    return pl.pallas_call(
        body,
        grid=(5,),
        in_specs=[pl.BlockSpec((2000, D), lambda i: (i, 0)),
                  pl.BlockSpec((D, D), lambda i: (0, 0))],
        out_specs=pl.BlockSpec((2000, D), lambda i: (i, 0)),
        out_shape=jax.ShapeDtypeStruct((N, D), jnp.float32),
    )(x, W)


def _add_relu(p0, p1):
    def body(a_ref, b_ref, o_ref):
        o_ref[...] = jnp.maximum(a_ref[...] + b_ref[...], 0.0)

    return pl.pallas_call(
        body,
        grid=(5,),
        in_specs=[pl.BlockSpec((2000, D), lambda i: (i, 0)),
                  pl.BlockSpec((2000, D), lambda i: (i, 0))],
        out_specs=pl.BlockSpec((2000, D), lambda i: (i, 0)),
        out_shape=jax.ShapeDtypeStruct((N, D), jnp.float32),
    )(p0, p1)


def _make_sc_spmm(chunks_per_tile):
    mesh = plsc.VectorSubcoreMesh(core_axis_name="c", subcore_axis_name="s")

    @functools.partial(
        pl.kernel,
        out_type=jax.ShapeDtypeStruct((NC, N_PAD, D), jnp.float32),
        mesh=mesh,
        scratch_types=[
            pltpu.VMEM((CHUNK,), jnp.int32),
            pltpu.VMEM((CHUNK,), jnp.int32),
            pltpu.VMEM((CHUNK,), jnp.float32),
            pltpu.VMEM((CHUNK, D), jnp.float32),
            pltpu.VMEM_SHARED((N_PAD, D), jnp.float32),
            pltpu.SemaphoreType.DMA,
        ],
    )
    def sc_spmm(h_hbm, src_hbm, dst_hbm, vals_hbm, out_hbm,
                src_v, dst_v, vals_v, rows_v, acc_sh, sem):
        c = lax.axis_index("c")
        s = lax.axis_index("s")
        wid = c * NS + s

        @pl.loop(0, CHUNK)
        def _(i):
            for cc in range(D // L):
                rows_v[i, pl.ds(cc * L, L)] = jnp.zeros((L,), jnp.float32)

        @pl.loop(0, ROWS_PER_TILE // CHUNK)
        def _(k):
            pltpu.sync_copy(
                rows_v, acc_sh.at[pl.ds(s * ROWS_PER_TILE + k * CHUNK, CHUNK)])

        plsc.subcore_barrier()

        @pl.loop(0, chunks_per_tile)
        def _(k):
            base = (wid * chunks_per_tile + k) * CHUNK
            pltpu.sync_copy(src_hbm.at[pl.ds(base, CHUNK)], src_v)
            pltpu.sync_copy(dst_hbm.at[pl.ds(base, CHUNK)], dst_v)
            pltpu.sync_copy(vals_hbm.at[pl.ds(base, CHUNK)], vals_v)
            pltpu.async_copy(h_hbm.at[src_v], rows_v, sem).wait()

            @pl.loop(0, CHUNK // L)
            def _(g):
                v16 = vals_v[pl.ds(g * L, L)]
                for l in range(L):
                    vb = lax.gather(
                        v16, jnp.full((L, 1), l, jnp.int32),
                        lax.GatherDimensionNumbers(
                            offset_dims=(), collapsed_slice_dims=(0,),
                            start_index_map=(0,)),
                        (1,), mode=lax.GatherScatterMode.PROMISE_IN_BOUNDS)
                    e = g * L + l
                    for cc in range(D // L):
                        sl = pl.ds(cc * L, L)
                        rows_v[e, sl] = rows_v[e, sl] * vb

            pltpu.sync_copy(rows_v, acc_sh.at[dst_v], add=True)

        plsc.subcore_barrier()

        @pl.loop(0, ROWS_PER_TILE // CHUNK)
        def _(k):
            r0 = s * ROWS_PER_TILE + k * CHUNK
            pltpu.sync_copy(acc_sh.at[pl.ds(r0, CHUNK)],
                            out_hbm.at[c].at[pl.ds(r0, CHUNK)])

    return sc_spmm


def kernel(x, edge_index, adj_vals, W):
    E = adj_vals.shape[0]
    chunks_per_tile = -(-E // (NW * CHUNK))
    e_pad = NW * CHUNK * chunks_per_tile
    pad = e_pad - E
    dst = jnp.pad(edge_index[0], (0, pad))
    src = jnp.pad(edge_index[1], (0, pad))
    vals = jnp.pad(adj_vals, (0, pad))

    h = _matmul(x, W)
    partials = _make_sc_spmm(chunks_per_tile)(h, src, dst, vals)
    return _add_relu(partials[0, :N], partials[1, :N])
